# baseline (speedup 1.0000x reference)
import functools

import jax
import jax.numpy as jnp
import numpy as np
from jax import lax
from jax.experimental import pallas as pl
from jax.experimental.pallas import tpu as pltpu
from jax.experimental.pallas import tpu_sc as plsc

_FIELD_DIMS = np.array([10000] * 39, dtype=np.int64)
_SEL = np.hstack((_FIELD_DIMS[:3], _FIELD_DIMS[4:8], _FIELD_DIMS[10:15],
                  _FIELD_DIMS[17:19], _FIELD_DIMS[21:24], _FIELD_DIMS[26:]))
_OFFSETS = np.array((0, *np.cumsum(_SEL)[:-1]), dtype=np.int32)

B = 4096
F = 30
FP = 32
D = 64
NC, NS, L = 2, 16, 16
NW = NC * NS
BW = B // NW
ROWS = 128
C = ROWS // FP
NCHUNK = BW // C
IPW = BW * FP


def _build(interpret=False):
  mesh = plsc.VectorSubcoreMesh(core_axis_name="c", subcore_axis_name="s",
                                num_cores=NC, num_subcores=NS)

  @functools.partial(
      pl.kernel,
      out_type=jax.ShapeDtypeStruct((B,), jnp.float32),
      mesh=mesh,
      interpret=interpret,
      compiler_params=pltpu.CompilerParams(needs_layout_passes=False,
                                           use_tc_tiling_on_sc=False),
      scratch_types=[
          pltpu.VMEM((IPW,), jnp.int32),
          pltpu.VMEM((2, ROWS, D), jnp.float32),
          pltpu.VMEM((BW,), jnp.float32),
          pltpu.VMEM((L,), jnp.float32),
          pltpu.SemaphoreType.DMA,
          pltpu.SemaphoreType.DMA,
      ],
  )
  def fm_kernel(idx_hbm, table_hbm, bias_hbm, out_hbm,
                idx_v, rows_v, out_v, bias_v, sem0, sem1):
    wid = lax.axis_index("s") * NC + lax.axis_index("c")
    sems = (sem0, sem1)

    pltpu.sync_copy(idx_hbm.at[pl.ds(wid * IPW, IPW)], idx_v)
    pltpu.sync_copy(bias_hbm, bias_v)

    def gather_start(c, buf):
      pltpu.async_copy(table_hbm.at[idx_v.at[pl.ds(c * ROWS, ROWS)]],
                       rows_v.at[buf], sems[buf])

    def gather_wait(c, buf):
      pltpu.make_async_copy(table_hbm.at[idx_v.at[pl.ds(c * ROWS, ROWS)]],
                            rows_v.at[buf], sems[buf]).wait()

    lanes = lax.iota(jnp.int32, L)

    def compute_chunk(c, buf, tvec):
      for bb in range(C):
        zero = jnp.zeros((L,), jnp.float32)

        def fbody(f, carry, _bb=bb):
          s0, s1, s2, s3, q = carry
          j = _bb * FP + f
          r0 = rows_v[buf, j, pl.ds(0, L)]
          r1 = rows_v[buf, j, pl.ds(L, L)]
          r2 = rows_v[buf, j, pl.ds(2 * L, L)]
          r3 = rows_v[buf, j, pl.ds(3 * L, L)]
          return (s0 + r0, s1 + r1, s2 + r2, s3 + r3,
                  q + r0 * r0 + r1 * r1 + r2 * r2 + r3 * r3)

        s0, s1, s2, s3, q = lax.fori_loop(0, F, fbody, (zero,) * 5)
        lin = jnp.sum(s0 + s1 + s2 + s3)
        sq = jnp.sum(s0 * s0 + s1 * s1 + s2 * s2 + s3 * s3)
        qs = jnp.sum(q)
        t = lin + 0.5 * (sq - qs)
        lane = (c * C + bb) % L
        tvec = jnp.where(lanes == lane, t, tvec)
      return tvec

    gather_start(0, 0)

    def pipe_body(i, tvec):
      c0 = 2 * i
      gather_start(c0 + 1, 1)
      gather_wait(c0, 0)
      tvec = compute_chunk(c0, 0, tvec)

      @pl.when(i < NCHUNK // 2 - 1)
      def _():
        gather_start(c0 + 2, 0)

      gather_wait(c0 + 1, 1)
      tvec = compute_chunk(c0 + 1, 1, tvec)

      @pl.when(i % 2 == 1)
      def _():
        out_v[pl.ds((i // 2) * L, L)] = tvec

      return tvec

    lax.fori_loop(0, NCHUNK // 2, pipe_body, jnp.zeros((L,), jnp.float32))

    bias_vec = bias_v[...]
    for k in range(BW // L):
      t = out_v[pl.ds(k * L, L)] + bias_vec
      out_v[pl.ds(k * L, L)] = 1.0 / (1.0 + jnp.exp(-t))

    pltpu.sync_copy(out_v, out_hbm.at[pl.ds(wid * BW, BW)])

  return fm_kernel


_FM_CACHE = []


def _get_fm():
  if not _FM_CACHE:
    _FM_CACHE.append(_build())
  return _FM_CACHE[0]


@jax.jit
def kernel(x, additional, emb_table, bias):
  del additional
  xs = jnp.concatenate((x[:, :3], x[:, 4:8], x[:, 10:15], x[:, 17:19],
                        x[:, 21:24], x[:, 26:]), axis=1)
  idx = xs.astype(jnp.int32) + jnp.asarray(_OFFSETS)
  idx = jnp.concatenate((idx, jnp.zeros((B, FP - F), jnp.int32)), axis=1)
  idx_flat = idx.reshape(B * FP)
  bias16 = jnp.broadcast_to(bias.astype(jnp.float32), (L,))
  return _get_fm()(idx_flat, emb_table, bias16)


# speedup vs baseline: 1.0026x; 1.0026x over previous
import functools

import jax
import jax.numpy as jnp
import numpy as np
from jax import lax
from jax.experimental import pallas as pl
from jax.experimental.pallas import tpu as pltpu
from jax.experimental.pallas import tpu_sc as plsc

_FIELD_DIMS = np.array([10000] * 39, dtype=np.int64)
_SEL = np.hstack((_FIELD_DIMS[:3], _FIELD_DIMS[4:8], _FIELD_DIMS[10:15],
                  _FIELD_DIMS[17:19], _FIELD_DIMS[21:24], _FIELD_DIMS[26:]))
_OFFSETS = np.array((0, *np.cumsum(_SEL)[:-1]), dtype=np.int32)

B = 4096
F = 30
FP = 32
D = 64
NC, NS, L = 2, 16, 16
NW = NC * NS
BW = B // NW
ROWS = 128
C = ROWS // FP
NCHUNK = BW // C
IPW = BW * FP


def _build(interpret=False):
  mesh = plsc.VectorSubcoreMesh(core_axis_name="c", subcore_axis_name="s",
                                num_cores=NC, num_subcores=NS)

  @functools.partial(
      pl.kernel,
      out_type=jax.ShapeDtypeStruct((B,), jnp.float32),
      mesh=mesh,
      interpret=interpret,
      compiler_params=pltpu.CompilerParams(needs_layout_passes=False,
                                           use_tc_tiling_on_sc=False),
      scratch_types=[
          pltpu.VMEM((NCHUNK, ROWS), jnp.int32),
          pltpu.VMEM((2, ROWS, D), jnp.float32),
          pltpu.VMEM((BW,), jnp.float32),
          pltpu.VMEM((L,), jnp.float32),
          pltpu.SemaphoreType.DMA,
          pltpu.SemaphoreType.DMA,
      ],
  )
  def fm_kernel(idx_hbm, table_hbm, bias_hbm, out_hbm,
                idx_v, rows_v, out_v, bias_v, sem0, sem1):
    wid = lax.axis_index("s") * NC + lax.axis_index("c")
    sems = (sem0, sem1)

    pltpu.sync_copy(idx_hbm.at[pl.ds(wid * NCHUNK, NCHUNK)], idx_v)
    pltpu.sync_copy(bias_hbm, bias_v)

    def gather_start(c, buf):
      pltpu.async_copy(table_hbm.at[idx_v.at[c]], rows_v.at[buf], sems[buf])

    def gather_wait(c, buf):
      pltpu.make_async_copy(table_hbm.at[idx_v.at[c]], rows_v.at[buf],
                            sems[buf]).wait()

    lanes = lax.iota(jnp.int32, L)

    def compute_chunk(c, buf, tvec):
      for bb in range(C):
        zero = jnp.zeros((L,), jnp.float32)

        def fbody(f, carry, _bb=bb):
          s0, s1, s2, s3, q = carry
          j = _bb * FP + f
          r0 = rows_v[buf, j, pl.ds(0, L)]
          r1 = rows_v[buf, j, pl.ds(L, L)]
          r2 = rows_v[buf, j, pl.ds(2 * L, L)]
          r3 = rows_v[buf, j, pl.ds(3 * L, L)]
          return (s0 + r0, s1 + r1, s2 + r2, s3 + r3,
                  q + r0 * r0 + r1 * r1 + r2 * r2 + r3 * r3)

        s0, s1, s2, s3, q = lax.fori_loop(0, F, fbody, (zero,) * 5)
        lin = jnp.sum(s0 + s1 + s2 + s3)
        sq = jnp.sum(s0 * s0 + s1 * s1 + s2 * s2 + s3 * s3)
        qs = jnp.sum(q)
        t = lin + 0.5 * (sq - qs)
        lane = (c * C + bb) % L
        tvec = jnp.where(lanes == lane, t, tvec)
      return tvec

    gather_start(0, 0)

    def pipe_body(i, tvec):
      c0 = 2 * i
      gather_start(c0 + 1, 1)
      gather_wait(c0, 0)
      tvec = compute_chunk(c0, 0, tvec)

      @pl.when(i < NCHUNK // 2 - 1)
      def _():
        gather_start(c0 + 2, 0)

      gather_wait(c0 + 1, 1)
      tvec = compute_chunk(c0 + 1, 1, tvec)

      @pl.when(i % 2 == 1)
      def _():
        out_v[pl.ds((i // 2) * L, L)] = tvec

      return tvec

    lax.fori_loop(0, NCHUNK // 2, pipe_body, jnp.zeros((L,), jnp.float32))

    bias_vec = bias_v[...]
    for k in range(BW // L):
      t = out_v[pl.ds(k * L, L)] + bias_vec
      out_v[pl.ds(k * L, L)] = 1.0 / (1.0 + jnp.exp(-t))

    pltpu.sync_copy(out_v, out_hbm.at[pl.ds(wid * BW, BW)])

  return fm_kernel


_FM_CACHE = []


def _get_fm():
  if not _FM_CACHE:
    _FM_CACHE.append(_build())
  return _FM_CACHE[0]


@jax.jit
def kernel(x, additional, emb_table, bias):
  del additional
  xs = jnp.concatenate((x[:, :3], x[:, 4:8], x[:, 10:15], x[:, 17:19],
                        x[:, 21:24], x[:, 26:]), axis=1)
  idx = xs.astype(jnp.int32) + jnp.asarray(_OFFSETS)
  idx = jnp.concatenate((idx, jnp.zeros((B, FP - F), jnp.int32)), axis=1)
  idx2d = idx.reshape(B * FP // ROWS, ROWS)
  bias16 = jnp.broadcast_to(bias.astype(jnp.float32), (L,))
  return _get_fm()(idx2d, emb_table, bias16)


# speedup vs baseline: 1.6718x; 1.6675x over previous
import functools

import jax
import jax.numpy as jnp
import numpy as np
from jax import lax
from jax.experimental import pallas as pl
from jax.experimental.pallas import tpu as pltpu
from jax.experimental.pallas import tpu_sc as plsc

_FIELD_DIMS = np.array([10000] * 39, dtype=np.int64)
_SEL = np.hstack((_FIELD_DIMS[:3], _FIELD_DIMS[4:8], _FIELD_DIMS[10:15],
                  _FIELD_DIMS[17:19], _FIELD_DIMS[21:24], _FIELD_DIMS[26:]))
_OFFSETS = np.array((0, *np.cumsum(_SEL)[:-1]), dtype=np.int32)

B = 4096
F = 30
FP = 32
D = 64
NC, NS, L = 2, 16, 16
NW = NC * NS
BW = B // NW
ROWS = 128
C = ROWS // FP
NCHUNK = BW // C
IPW = BW * FP


def _build(interpret=False):
  mesh = plsc.VectorSubcoreMesh(core_axis_name="c", subcore_axis_name="s",
                                num_cores=NC, num_subcores=NS)

  @functools.partial(
      pl.kernel,
      out_type=jax.ShapeDtypeStruct((B,), jnp.float32),
      mesh=mesh,
      interpret=interpret,
      compiler_params=pltpu.CompilerParams(needs_layout_passes=False,
                                           use_tc_tiling_on_sc=False),
      scratch_types=[
          pltpu.VMEM((NCHUNK, ROWS), jnp.int32),
          pltpu.VMEM((2, ROWS, D), jnp.float32),
          pltpu.VMEM((BW,), jnp.float32),
          pltpu.VMEM((L,), jnp.float32),
          pltpu.SemaphoreType.DMA,
          pltpu.SemaphoreType.DMA,
      ],
  )
  def fm_kernel(idx_hbm, table_hbm, bias_hbm, out_hbm,
                idx_v, rows_v, out_v, bias_v, sem0, sem1):
    wid = lax.axis_index("s") * NC + lax.axis_index("c")
    sems = (sem0, sem1)

    pltpu.sync_copy(idx_hbm.at[pl.ds(wid * NCHUNK, NCHUNK)], idx_v)
    pltpu.sync_copy(bias_hbm, bias_v)

    def gather_start(c, buf):
      pltpu.async_copy(table_hbm.at[idx_v.at[c]], rows_v.at[buf], sems[buf])

    def gather_wait(c, buf):
      pltpu.make_async_copy(table_hbm.at[idx_v.at[c]], rows_v.at[buf],
                            sems[buf]).wait()

    lanes = lax.iota(jnp.int32, L)

    def compute_chunk(c, buf, tvec):
      for bb in range(C):
        zero = jnp.zeros((L,), jnp.float32)

        def fbody(f, carry, _bb=bb):
          s0, s1, s2, s3, q = carry
          j = _bb * FP + f
          r0 = rows_v[buf, j, pl.ds(0, L)]
          r1 = rows_v[buf, j, pl.ds(L, L)]
          r2 = rows_v[buf, j, pl.ds(2 * L, L)]
          r3 = rows_v[buf, j, pl.ds(3 * L, L)]
          return (s0 + r0, s1 + r1, s2 + r2, s3 + r3,
                  q + r0 * r0 + r1 * r1 + r2 * r2 + r3 * r3)

        s0, s1, s2, s3, q = lax.fori_loop(0, F, fbody, (zero,) * 5)
        lin = jnp.sum(s0 + s1 + s2 + s3)
        sq = jnp.sum(s0 * s0 + s1 * s1 + s2 * s2 + s3 * s3)
        qs = jnp.sum(q)
        t = lin + 0.5 * (sq - qs)
        lane = (c * C + bb) % L
        tvec = jnp.where(lanes == lane, t, tvec)
      return tvec

    gather_start(0, 0)

    def pipe_body(i, tvec):
      c0 = 2 * i
      gather_start(c0 + 1, 1)
      gather_wait(c0, 0)
      tvec = compute_chunk(c0, 0, tvec)

      @pl.when(i < NCHUNK // 2 - 1)
      def _():
        gather_start(c0 + 2, 0)

      gather_wait(c0 + 1, 1)
      tvec = compute_chunk(c0 + 1, 1, tvec)

      @pl.when(i % 2 == 1)
      def _():
        out_v[pl.ds((i // 2) * L, L)] = tvec

      return tvec

    lax.fori_loop(0, NCHUNK // 2, pipe_body, jnp.zeros((L,), jnp.float32))

    bias_vec = bias_v[...]
    for k in range(BW // L):
      t = out_v[pl.ds(k * L, L)] + bias_vec
      out_v[pl.ds(k * L, L)] = 1.0 / (1.0 + jnp.exp(-t))

    pltpu.sync_copy(out_v, out_hbm.at[pl.ds(wid * BW, BW)])

  return fm_kernel


_FM_CACHE = []


def _get_fm():
  if not _FM_CACHE:
    _FM_CACHE.append(_build())
  return _FM_CACHE[0]


@jax.jit
def kernel(x, additional, emb_table, bias):
  del additional
  xs = jnp.concatenate((x[:, :3], x[:, 4:8], x[:, 10:15], x[:, 17:19],
                        x[:, 21:24], x[:, 26:]), axis=1)
  idx = xs.astype(jnp.int32) + jnp.asarray(_OFFSETS)
  pad = jnp.arange(B * (FP - F), dtype=jnp.int32).reshape(B, FP - F)
  idx = jnp.concatenate((idx, pad), axis=1)
  idx2d = idx.reshape(B * FP // ROWS, ROWS)
  bias16 = jnp.broadcast_to(bias.astype(jnp.float32), (L,))
  return _get_fm()(idx2d, emb_table, bias16)
